# Initial kernel scaffold; baseline (speedup 1.0000x reference)
import functools

import jax
import jax.numpy as jnp
from jax import lax
from jax.experimental import pallas as pl
from jax.experimental.pallas import tpu as pltpu
from jax.experimental.pallas import tpu_sc as plsc

N_NODES = 10000
D_IN = 128
D_H = 64
N_EDGES = 320000

NC = 2
NS = 16
NW = NC * NS

EDGE_BATCH = 128
BATCHES_PER_W = 79
E_PAD = NW * BATCHES_PER_W * EDGE_BATCH
ACC_ROWS = 10240
ZERO_ROWS = ACC_ROWS // NS
OUT_ROWS = N_NODES // NS

_sc_mesh = plsc.VectorSubcoreMesh(core_axis_name="c", subcore_axis_name="s")


@functools.partial(
    pl.kernel,
    mesh=_sc_mesh,
    out_type=jax.ShapeDtypeStruct((NC, N_NODES, D_H), jnp.float32),
    scratch_types=[
        pltpu.VMEM((BATCHES_PER_W, EDGE_BATCH), jnp.int32),
        pltpu.VMEM((BATCHES_PER_W, EDGE_BATCH), jnp.int32),
        pltpu.VMEM((EDGE_BATCH, D_H), jnp.float32),
        pltpu.VMEM((EDGE_BATCH, D_H), jnp.float32),
        pltpu.VMEM_SHARED((ACC_ROWS, D_H), jnp.float32),
        pltpu.SemaphoreType.DMA,
        pltpu.SemaphoreType.DMA,
    ],
)
def _segment_sum_sc(z_hbm, src_hbm, dst_hbm, zeros_hbm, out_hbm,
                    src_v, dst_v, buf0, buf1, acc, sem0, sem1):
    c = lax.axis_index("c")
    s = lax.axis_index("s")
    wid = s * NC + c

    pltpu.sync_copy(zeros_hbm, acc.at[pl.ds(s * ZERO_ROWS, ZERO_ROWS)])
    pltpu.sync_copy(src_hbm.at[wid], src_v)
    pltpu.sync_copy(dst_hbm.at[wid], dst_v)
    plsc.subcore_barrier()

    def pair_body(i, _):
        j = i * 2
        nxt = pltpu.async_copy(z_hbm.at[src_v.at[j + 1]], buf1, sem1)
        pltpu.async_copy(z_hbm.at[src_v.at[j]], buf0, sem0).wait()
        pltpu.sync_copy(buf0, acc.at[dst_v.at[j]], add=True)
        nxt.wait()
        pltpu.sync_copy(buf1, acc.at[dst_v.at[j + 1]], add=True)
        return 0
    lax.fori_loop(0, BATCHES_PER_W // 2, pair_body, 0)
    j_last = BATCHES_PER_W - 1
    pltpu.async_copy(z_hbm.at[src_v.at[j_last]], buf0, sem0).wait()
    pltpu.sync_copy(buf0, acc.at[dst_v.at[j_last]], add=True)

    plsc.subcore_barrier()
    pltpu.sync_copy(acc.at[pl.ds(s * OUT_ROWS, OUT_ROWS)],
                    out_hbm.at[c, pl.ds(s * OUT_ROWS, OUT_ROWS)])


def _mm_body(x_ref, w_ref, o_ref):
    o_ref[...] = jnp.dot(x_ref[...], w_ref[...],
                         preferred_element_type=jnp.float32)


def _fuse_body(z_ref, a_ref, b_ref, w_ref, o_ref):
    h = jnp.maximum(z_ref[...] + a_ref[0] + a_ref[1] + b_ref[...], 0.0)
    o_ref[...] = jnp.dot(h, w_ref[...], preferred_element_type=jnp.float32)


def _head_body(z_ref, a_ref, b2_ref, w3_ref, b3_ref, w4_ref, b4_ref, o_ref):
    h2 = jnp.maximum(z_ref[...] + a_ref[0] + a_ref[1] + b2_ref[...], 0.0)
    h3 = jnp.maximum(jnp.dot(h2, w3_ref[...],
                             preferred_element_type=jnp.float32) + b3_ref[...],
                     0.0)
    o_ref[...] = jnp.dot(h3, w4_ref[...],
                         preferred_element_type=jnp.float32) + b4_ref[...]


_mm = pl.pallas_call(
    _mm_body, out_shape=jax.ShapeDtypeStruct((N_NODES, D_H), jnp.float32))

_fuse = pl.pallas_call(
    _fuse_body, out_shape=jax.ShapeDtypeStruct((N_NODES, D_H), jnp.float32))

_head = pl.pallas_call(
    _head_body, out_shape=jax.ShapeDtypeStruct((N_NODES, 1), jnp.float32))


def kernel(x, edge_index, batch, W1, b1, W2, b2, W3, b3, W4, b4):
    del batch
    x = x.astype(jnp.float32)
    src = edge_index[0].astype(jnp.int32)
    dst = edge_index[1].astype(jnp.int32)

    pad = E_PAD - N_EDGES
    src_p = jnp.concatenate([src, jnp.zeros((pad,), jnp.int32)])
    dst_p = jnp.concatenate([dst, jnp.full((pad,), N_NODES, jnp.int32)])
    src3 = src_p.reshape(NW, BATCHES_PER_W, EDGE_BATCH)
    dst3 = dst_p.reshape(NW, BATCHES_PER_W, EDGE_BATCH)
    zeros = jnp.zeros((ZERO_ROWS, D_H), jnp.float32)

    b1r = b1.reshape(1, D_H)
    b2r = b2.reshape(1, D_H)
    b3r = b3.reshape(1, 16)
    b4r = b4.reshape(1, 1)

    z1 = _mm(x, W1)
    a1 = _segment_sum_sc(z1, src3, dst3, zeros)
    z2 = _fuse(z1, a1, b1r, W2)
    a2 = _segment_sum_sc(z2, src3, dst3, zeros)
    out = _head(z2, a2, b2r, W3, b3r, W4, b4r)
    return out


# speedup vs baseline: 7.4398x; 7.4398x over previous
import functools

import jax
import jax.numpy as jnp
from jax import lax
from jax.experimental import pallas as pl
from jax.experimental.pallas import tpu as pltpu
from jax.experimental.pallas import tpu_sc as plsc

N_NODES = 10000
D_IN = 128
D_H = 64
N_EDGES = 320000

NC = 2
NS = 16
NW = NC * NS

EDGE_BATCH = 128
BATCHES_PER_W = 79
E_PAD = NW * BATCHES_PER_W * EDGE_BATCH
ACC_ROWS = 10240
ZERO_ROWS = ACC_ROWS // NS
OUT_ROWS = N_NODES // NS

@functools.cache
def _build_segment_sum_sc():
    mesh = plsc.VectorSubcoreMesh(core_axis_name="c", subcore_axis_name="s")
    return functools.partial(
        pl.kernel,
        mesh=mesh,
        compiler_params=pltpu.CompilerParams(use_tc_tiling_on_sc=False),
        out_type=jax.ShapeDtypeStruct((NC, ACC_ROWS, D_H), jnp.float32),
        scratch_types=[
            pltpu.VMEM((BATCHES_PER_W, EDGE_BATCH), jnp.int32),
            pltpu.VMEM((BATCHES_PER_W, EDGE_BATCH), jnp.int32),
            pltpu.VMEM((EDGE_BATCH, D_H), jnp.float32),
            pltpu.VMEM((EDGE_BATCH, D_H), jnp.float32),
            pltpu.VMEM_SHARED((ACC_ROWS, D_H), jnp.float32),
            pltpu.SemaphoreType.DMA,
            pltpu.SemaphoreType.DMA,
        ],
    )(_segment_sum_sc_body)


def _segment_sum_sc_body(z_hbm, src_hbm, dst_hbm, zeros_hbm, out_hbm,
                    src_v, dst_v, buf0, buf1, acc, sem0, sem1):
    c = lax.axis_index("c")
    s = lax.axis_index("s")
    wid = s * NC + c

    pltpu.sync_copy(zeros_hbm, acc.at[pl.ds(s * ZERO_ROWS, ZERO_ROWS)])
    pltpu.sync_copy(src_hbm.at[wid], src_v)
    pltpu.sync_copy(dst_hbm.at[wid], dst_v)
    plsc.subcore_barrier()

    def pair_body(i, _):
        j = i * 2
        nxt = pltpu.async_copy(z_hbm.at[src_v.at[j + 1]], buf1, sem1)
        pltpu.async_copy(z_hbm.at[src_v.at[j]], buf0, sem0).wait()
        pltpu.sync_copy(buf0, acc.at[dst_v.at[j]], add=True)
        nxt.wait()
        pltpu.sync_copy(buf1, acc.at[dst_v.at[j + 1]], add=True)
        return 0
    lax.fori_loop(0, BATCHES_PER_W // 2, pair_body, 0)
    j_last = BATCHES_PER_W - 1
    pltpu.async_copy(z_hbm.at[src_v.at[j_last]], buf0, sem0).wait()
    pltpu.sync_copy(buf0, acc.at[dst_v.at[j_last]], add=True)

    plsc.subcore_barrier()
    pltpu.sync_copy(acc.at[pl.ds(s * ZERO_ROWS, ZERO_ROWS)],
                    out_hbm.at[c, pl.ds(s * ZERO_ROWS, ZERO_ROWS)])


def _mm_body(x_ref, w_ref, o_ref):
    o_ref[...] = jnp.dot(x_ref[...], w_ref[...],
                         preferred_element_type=jnp.float32)


def _fuse_body(z_ref, a_ref, b_ref, w_ref, o_ref):
    h = jnp.maximum(z_ref[...] + a_ref[0] + a_ref[1] + b_ref[...], 0.0)
    o_ref[...] = jnp.dot(h, w_ref[...], preferred_element_type=jnp.float32)


def _head_body(z_ref, a_ref, b2_ref, w3_ref, b3_ref, w4_ref, b4_ref, o_ref):
    h2 = jnp.maximum(z_ref[...] + a_ref[0] + a_ref[1] + b2_ref[...], 0.0)
    h3 = jnp.maximum(jnp.dot(h2, w3_ref[...],
                             preferred_element_type=jnp.float32) + b3_ref[...],
                     0.0)
    o_ref[...] = jnp.dot(h3, w4_ref[...],
                         preferred_element_type=jnp.float32) + b4_ref[...]


_mm = pl.pallas_call(
    _mm_body, out_shape=jax.ShapeDtypeStruct((N_NODES, D_H), jnp.float32))

_fuse = pl.pallas_call(
    _fuse_body, out_shape=jax.ShapeDtypeStruct((N_NODES, D_H), jnp.float32))

_head = pl.pallas_call(
    _head_body, out_shape=jax.ShapeDtypeStruct((N_NODES, 1), jnp.float32))


def kernel(x, edge_index, batch, W1, b1, W2, b2, W3, b3, W4, b4):
    del batch
    x = x.astype(jnp.float32)
    src = edge_index[0].astype(jnp.int32)
    dst = edge_index[1].astype(jnp.int32)

    pad = E_PAD - N_EDGES
    src_p = jnp.concatenate([src, jnp.zeros((pad,), jnp.int32)])
    dst_p = jnp.concatenate([dst, jnp.full((pad,), N_NODES, jnp.int32)])
    src3 = src_p.reshape(NW, BATCHES_PER_W, EDGE_BATCH)
    dst3 = dst_p.reshape(NW, BATCHES_PER_W, EDGE_BATCH)
    zeros = jnp.zeros((ZERO_ROWS, D_H), jnp.float32)

    b1r = b1.reshape(1, D_H)
    b2r = b2.reshape(1, D_H)
    b3r = b3.reshape(1, 16)
    b4r = b4.reshape(1, 1)

    seg_sum = _build_segment_sum_sc()
    z1 = _mm(x, W1)
    a1 = seg_sum(z1, src3, dst3, zeros)[:, :N_NODES]
    z2 = _fuse(z1, a1, b1r, W2)
    a2 = seg_sum(z2, src3, dst3, zeros)[:, :N_NODES]
    out = _head(z2, a2, b2r, W3, b3r, W4, b4r)
    return out
